# Initial kernel scaffold; baseline (speedup 1.0000x reference)
import jax
import jax.numpy as jnp
from jax.experimental import pallas as pl


def kernel(x, pos, batch, W, b):
    raise NotImplementedError("write your pallas kernel here")


# speedup vs baseline: 14.8725x; 14.8725x over previous
import functools

import jax
import jax.numpy as jnp
import numpy as np
from jax import lax
from jax.experimental import pallas as pl
from jax.experimental.pallas import tpu as pltpu
from jax.experimental.pallas import tpu_sc as plsc

N = 10000
D = 128
NSAMP = 2500
K = 64
R2_BITS = 0x3D23D70A
NTILES = 32
QPT = 80
NQPAD = NTILES * QPT
CAND_CAP = 1024
NCH = N // 16


def _fps_body(px_ref, py_ref, pz_ref, idx_ref):
    px = px_ref[...]
    py = py_ref[...]
    pz = pz_ref[...]
    rows = lax.broadcasted_iota(jnp.int32, (80, 128), 0)
    cols = lax.broadcasted_iota(jnp.int32, (80, 128), 1)
    lin = rows * 128 + cols
    lin_out = (
        lax.broadcasted_iota(jnp.int32, (20, 128), 0) * 128
        + lax.broadcasted_iota(jnp.int32, (20, 128), 1)
    )
    sx = px[0, 0]
    sy = py[0, 0]
    sz = pz[0, 0]
    dx = px - sx
    dy = py - sy
    dz = pz - sz
    dists = jnp.where(lin < N, dx * dx + dy * dy + dz * dz, -1.0)

    def body(t, carry):
        dists, idx_buf = carry
        m = jnp.max(dists)
        nxt = jnp.min(jnp.where(dists == m, lin, jnp.int32(2**30)))
        idx_buf = jnp.where(lin_out == t, nxt, idx_buf)
        sel = lin == nxt
        sx = jnp.sum(jnp.where(sel, px, 0.0))
        sy = jnp.sum(jnp.where(sel, py, 0.0))
        sz = jnp.sum(jnp.where(sel, pz, 0.0))
        dx = px - sx
        dy = py - sy
        dz = pz - sz
        d = dx * dx + dy * dy + dz * dz
        return jnp.minimum(dists, d), idx_buf

    _, idx_buf = lax.fori_loop(
        1, NSAMP, body, (dists, jnp.zeros((20, 128), jnp.int32))
    )
    idx_ref[...] = idx_buf


def _fps(px, py, pz):
    return pl.pallas_call(
        _fps_body,
        out_shape=jax.ShapeDtypeStruct((20, 128), jnp.int32),
    )(px, py, pz)


def _gemm_body(a_ref, w_ref, g_ref):
    g_ref[...] = jnp.dot(a_ref[...], w_ref[...], preferred_element_type=jnp.float32)


def _gemm(a, w):
    return pl.pallas_call(
        _gemm_body,
        grid=(10,),
        in_specs=[
            pl.BlockSpec((1000, 136), lambda i: (i, 0)),
            pl.BlockSpec((136, 128), lambda i: (0, 0)),
        ],
        out_specs=pl.BlockSpec((1000, 128), lambda i: (i, 0)),
        out_shape=jax.ShapeDtypeStruct((N, 128), jnp.float32),
    )(a, w)


def _make_sc_agg():
  kernel_deco = functools.partial(
    pl.kernel,
    out_type=jax.ShapeDtypeStruct((NQPAD, 128), jnp.float32),
    mesh=plsc.VectorSubcoreMesh(
        core_axis_name="c", subcore_axis_name="s", num_cores=2, num_subcores=16
    ),
    compiler_params=pltpu.CompilerParams(needs_layout_passes=False),
    scratch_types=[
        pltpu.VMEM((N + 16,), jnp.float32),
        pltpu.VMEM((N + 16,), jnp.float32),
        pltpu.VMEM((N + 16,), jnp.float32),
        pltpu.VMEM((NSAMP + 16,), jnp.int32),
        pltpu.VMEM((3, 128), jnp.float32),
        pltpu.VMEM((128,), jnp.float32),
        pltpu.VMEM((CAND_CAP + 48,), jnp.float32),
        pltpu.VMEM((CAND_CAP + 48,), jnp.int32),
        pltpu.VMEM((K + 16,), jnp.int32),
        pltpu.VMEM((K, 128), jnp.float32),
        pltpu.VMEM((QPT, 128), jnp.float32),
        pltpu.SemaphoreType.DMA,
    ],
  )

  @kernel_deco
  def _sc_agg(
      px_h, py_h, pz_h, idx_h, wp_h, b_h, g_h, out_h,
      px_v, py_v, pz_v, idx_v, wp_v, b_v, cval, cidx, sel, rows, out_v, sem,
  ):
      wid = lax.axis_index("s") * 2 + lax.axis_index("c")
      base = wid * QPT
      nq = jnp.maximum(jnp.minimum(jnp.int32(NSAMP) - base, jnp.int32(QPT)), 0)
      pltpu.sync_copy(px_h, px_v)
      pltpu.sync_copy(py_h, py_v)
      pltpu.sync_copy(pz_h, pz_v)
      pltpu.sync_copy(idx_h, idx_v)
      pltpu.sync_copy(wp_h, wp_v)
      pltpu.sync_copy(b_h, b_v)

      lane = lax.iota(jnp.int32, 16)

      def per_query(i, _):
          q = base + i
          qi = idx_v[pl.ds(q, 16)][0]
          qx = px_v[pl.ds(qi, 16)][0]
          qy = py_v[pl.ds(qi, 16)][0]
          qz = pz_v[pl.ds(qi, 16)][0]

          def chunk(c, off):
              s = c * 16
              dx = px_v[pl.ds(s, 16)] - qx
              dy = py_v[pl.ds(s, 16)] - qy
              dz = pz_v[pl.ds(s, 16)] - qz
              d2 = dx * dx + dy * dy + dz * dz
              m = d2 <= jnp.float32(0.04)
              offc = jnp.minimum(off, jnp.int32(CAND_CAP))
              plsc.store_compressed(cval.at[pl.ds(offc, 16)], d2, mask=m)
              plsc.store_compressed(cidx.at[pl.ds(offc, 16)], lane + s, mask=m)
              return off + jnp.sum(m.astype(jnp.int32))

          ncand = lax.fori_loop(0, NCH, chunk, jnp.int32(0))
          ncand = jnp.minimum(ncand, jnp.int32(CAND_CAP))
          cval[pl.ds(ncand, 16)] = jnp.zeros((16,), jnp.float32) + jnp.float32(1e30)

          nch = (ncand + 15) // 16

          def bis(_, lohi):
              lo, hi = lohi
              mid = (lo + hi) // 2
              mid_f = lax.bitcast_convert_type(mid, jnp.float32)

              def cnt_chunk(c, acc):
                  v = cval[pl.ds(c * 16, 16)]
                  return acc + jnp.where(v <= mid_f, 1, 0).astype(jnp.int32)

              accv = lax.fori_loop(0, nch, cnt_chunk, jnp.zeros((16,), jnp.int32))
              cnt = jnp.sum(accv)
              big = cnt >= K
              return jnp.where(big, lo, mid + 1), jnp.where(big, mid, hi)

          _, t = lax.fori_loop(0, 30, bis, (jnp.int32(0), jnp.int32(R2_BITS)))

          for j in range((K + 16) // 16):
              sel[pl.ds(16 * j, 16)] = jnp.zeros((16,), jnp.int32) + qi

          t_f = lax.bitcast_convert_type(t, jnp.float32)

          def selchunk(c, off):
              v = cval[pl.ds(c * 16, 16)]
              m = v <= t_f
              offc = jnp.minimum(off, jnp.int32(K))
              plsc.store_compressed(
                  sel.at[pl.ds(offc, 16)], cidx[pl.ds(c * 16, 16)], mask=m
              )
              return off + jnp.sum(m.astype(jnp.int32))

          lax.fori_loop(0, nch, selchunk, jnp.int32(0))

          pltpu.async_copy(g_h.at[sel.at[pl.ds(0, K)]], rows, sem).wait()

          accs = tuple(rows[0, pl.ds(16 * k, 16)] for k in range(8))

          def rowmax(j, accs):
              return tuple(
                  jnp.maximum(a, rows[j, pl.ds(16 * k, 16)])
                  for k, a in enumerate(accs)
              )

          accs = lax.fori_loop(1, K, rowmax, accs)

          for k in range(8):
              ck = (
                  qx * wp_v[0, pl.ds(16 * k, 16)]
                  + qy * wp_v[1, pl.ds(16 * k, 16)]
                  + qz * wp_v[2, pl.ds(16 * k, 16)]
              )
              o = jnp.maximum(accs[k] - ck + b_v[pl.ds(16 * k, 16)], 0.0)
              out_v[i, pl.ds(16 * k, 16)] = o
          return 0

      lax.fori_loop(0, nq, per_query, 0)
      pltpu.sync_copy(out_v, out_h.at[pl.ds(base, QPT)])


  return _sc_agg

_SC_AGG = None
def _sc_agg_call(*args):
    global _SC_AGG
    if _SC_AGG is None:
        _SC_AGG = _make_sc_agg()
    return _SC_AGG(*args)


@jax.jit
def kernel(x, pos, batch, W, b):
    pos_t = pos.T
    pos_pad = jnp.concatenate(
        [pos_t, jnp.zeros((3, 80 * 128 - N), jnp.float32)], axis=1
    )
    px = pos_pad[0].reshape(80, 128)
    py = pos_pad[1].reshape(80, 128)
    pz = pos_pad[2].reshape(80, 128)

    idx_flat = _fps(px, py, pz).reshape(-1)
    idx = idx_flat[:NSAMP]

    a = jnp.concatenate([x, pos, jnp.zeros((N, 5), jnp.float32)], axis=1)
    w_pad = jnp.concatenate([W, jnp.zeros((5, 128), jnp.float32)], axis=0)
    g = _gemm(a, w_pad)

    wp = W[128:131]
    pos_sc = pos_pad[:, : N + 16]
    x_out_pad = _sc_agg_call(
        pos_sc[0], pos_sc[1], pos_sc[2], idx_flat[: NSAMP + 16], wp, b, g
    )
    x_out = x_out_pad[:NSAMP]

    pos_q = pos[idx]
    return (x_out, pos_q, batch[idx])
